# baseline (speedup 1.0000x reference)
import functools

import jax
import jax.numpy as jnp
from jax import lax
from jax.experimental import pallas as pl
from jax.experimental.pallas import tpu as pltpu
from jax.experimental.pallas import tpu_sc as plsc

_N = 320000
_D = 128
_S = 10000
_BETA = 0.8
_W = 128
_B = 2560
_NB = _N // _B
_SEG_PAD = 10240

_NW = 32
_RPW = _N // _NW
_SPW = _SEG_PAD // _NW
_L = 16


def _sc_partial_body(ids_hbm, ts_hbm, out_hbm, ids_v, ts_v, acc_v):
    c = lax.axis_index("c")
    s = lax.axis_index("s")
    w = s * 2 + c
    base = w * _RPW

    pltpu.sync_copy(ids_hbm.at[pl.ds(base, _RPW)], ids_v)
    pltpu.sync_copy(ts_hbm.at[pl.ds(base, _RPW)], ts_v)

    zeros16 = jnp.zeros((_L,), jnp.float32)

    def zbody(i, carry):
        acc_v[pl.ds(i * _L, _L)] = zeros16
        return carry

    lax.fori_loop(0, _SEG_PAD // _L, zbody, 0)

    lane = lax.broadcasted_iota(jnp.int32, (_L,), 0)

    def gbody(g, carry):
        ids16 = ids_v[pl.ds(g * _L, _L)]
        t16 = ts_v[pl.ds(g * _L, _L)]
        for d in (1, 2, 4, 8):
            fidx = jnp.minimum(lane + d, _L - 1)
            bidx = jnp.maximum(lane - d, 0)
            tf = t16.at[fidx].get(mode="promise_in_bounds")
            idf = ids16.at[fidx].get(mode="promise_in_bounds")
            tb = t16.at[bidx].get(mode="promise_in_bounds")
            idb = ids16.at[bidx].get(mode="promise_in_bounds")
            t16 = jnp.maximum(t16, jnp.where(idf == ids16, tf, 0.0))
            t16 = jnp.maximum(t16, jnp.where(idb == ids16, tb, 0.0))
        cur = plsc.load_gather(acc_v, [ids16])
        plsc.store_scatter(acc_v, [ids16], jnp.maximum(cur, t16))
        return carry

    lax.fori_loop(0, _RPW // _L, gbody, 0)

    pltpu.sync_copy(acc_v, out_hbm.at[pl.ds(w * _SEG_PAD, _SEG_PAD)])


_sc_partial = functools.partial(
    pl.kernel,
    out_type=jax.ShapeDtypeStruct((_NW * _SEG_PAD,), jnp.float32),
    mesh=plsc.VectorSubcoreMesh(core_axis_name="c", subcore_axis_name="s"),
    compiler_params=pltpu.CompilerParams(needs_layout_passes=False),
    scratch_types=[
        pltpu.VMEM((_RPW,), jnp.int32),
        pltpu.VMEM((_RPW,), jnp.float32),
        pltpu.VMEM((_SEG_PAD,), jnp.float32),
    ],
)(_sc_partial_body)


def _sc_fold_body(parts_hbm, out_hbm, buf_v, acc_v, sem):
    c = lax.axis_index("c")
    s = lax.axis_index("s")
    w = s * 2 + c
    base = w * _SPW

    descs = [
        pltpu.async_copy(
            parts_hbm.at[pl.ds(r * _SEG_PAD + base, _SPW)],
            buf_v.at[pl.ds(r * _SPW, _SPW)], sem)
        for r in range(_NW)
    ]
    for d in descs:
        d.wait()

    def fbody(i, carry):
        v = buf_v[pl.ds(i * _L, _L)]
        for r in range(1, _NW):
            v = jnp.maximum(v, buf_v[pl.ds(r * _SPW + i * _L, _L)])
        acc_v[pl.ds(i * _L, _L)] = v
        return carry

    lax.fori_loop(0, _SPW // _L, fbody, 0)

    pltpu.sync_copy(acc_v, out_hbm.at[pl.ds(base, _SPW)])


_sc_fold = functools.partial(
    pl.kernel,
    out_type=jax.ShapeDtypeStruct((_SEG_PAD,), jnp.float32),
    mesh=plsc.VectorSubcoreMesh(core_axis_name="c", subcore_axis_name="s"),
    compiler_params=pltpu.CompilerParams(needs_layout_passes=False),
    scratch_types=[
        pltpu.VMEM((_NW * _SPW,), jnp.float32),
        pltpu.VMEM((_SPW,), jnp.float32),
        pltpu.SemaphoreType.DMA,
    ],
)(_sc_fold_body)


def _tc_body(ids_ref, ts_ref, msg_ref, smax_ref, agg_ref, wsum_ref):
    j = pl.program_id(0)

    id_row = ids_ref[0]
    t_row = ts_ref[0]

    base0 = jnp.minimum((jnp.min(id_row) // 8) * 8, _SEG_PAD - _W)
    nw = (jnp.max(id_row) - base0) // _W + 1

    seg_iota = lax.broadcasted_iota(jnp.int32, (_W, 1), 0)

    @pl.when(j == 0)
    def _init():
        wsum_ref[...] = jnp.zeros_like(wsum_ref)
        agg_ref[...] = jnp.zeros_like(agg_ref)

    msg = msg_ref[...]
    ones_col = jnp.ones((_B, 1), jnp.float32)

    def body(w, carry):
        start = base0 + w * _W
        base = jnp.minimum(start, _SEG_PAD - _W)
        rel = jnp.where(id_row >= start, id_row - base, -1)
        eq = rel == seg_iota
        onehot = jnp.where(eq, 1.0, 0.0)
        smax_win = smax_ref[pl.ds(base, _W), :]
        last_t = jnp.sum(onehot * smax_win, axis=0, keepdims=True)
        arg = jnp.minimum(_BETA * (t_row - last_t), 0.0)
        w_row = jnp.exp(arg)
        wo = onehot * w_row
        wsum_ref[pl.ds(base, _W), :] += jnp.dot(
            wo, ones_col, preferred_element_type=jnp.float32)
        agg_ref[pl.ds(base, _W), :] += jnp.dot(
            wo, msg, preferred_element_type=jnp.float32)
        return carry

    lax.fori_loop(0, nw, body, 0)

    @pl.when(j == _NB - 1)
    def _finish():
        wsum = wsum_ref[...]
        inv = jnp.where(wsum > 0.0, 1.0 / wsum, 0.0)
        agg_ref[...] = agg_ref[...] * inv


@functools.partial(jax.jit, static_argnames=("interpret",))
def _run(node_ids, messages, timestamps, interpret=False):
    ids = node_ids.astype(jnp.int32)
    parts = _sc_partial(ids, timestamps)
    seg_max_col = _sc_fold(parts).reshape(_SEG_PAD, 1)

    ids3 = ids.reshape(_NB, 1, _B)
    ts3 = timestamps.reshape(_NB, 1, _B)

    out = pl.pallas_call(
        _tc_body,
        grid=(_NB,),
        in_specs=[
            pl.BlockSpec((1, 1, _B), lambda j: (j, 0, 0)),
            pl.BlockSpec((1, 1, _B), lambda j: (j, 0, 0)),
            pl.BlockSpec((_B, _D), lambda j: (j, 0)),
            pl.BlockSpec((_SEG_PAD, 1), lambda j: (0, 0)),
        ],
        out_specs=[
            pl.BlockSpec((_SEG_PAD, _D), lambda j: (0, 0)),
            pl.BlockSpec((_SEG_PAD, 1), lambda j: (0, 0)),
        ],
        out_shape=[
            jax.ShapeDtypeStruct((_SEG_PAD, _D), jnp.float32),
            jax.ShapeDtypeStruct((_SEG_PAD, 1), jnp.float32),
        ],
        interpret=interpret,
    )(ids3, ts3, messages, seg_max_col)

    agg, wsum = out
    return (agg[:_S], seg_max_col[:_S, 0], wsum[:_S, 0] > 0.0)


def kernel(node_ids, messages, timestamps):
    return _run(node_ids, messages, timestamps)


# speedup vs baseline: 1.0670x; 1.0670x over previous
import functools

import jax
import jax.numpy as jnp
from jax import lax
from jax.experimental import pallas as pl
from jax.experimental.pallas import tpu as pltpu
from jax.experimental.pallas import tpu_sc as plsc

_N = 320000
_D = 128
_S = 10000
_BETA = 0.8
_W = 128
_B = 3200
_NB = _N // _B
_SEG_PAD = 10240

_NW = 32
_RPW = _N // _NW
_SPW = _SEG_PAD // _NW
_L = 16


def _sc_partial_body(ids_hbm, ts_hbm, out_hbm, ids_v, ts_v, acc_v):
    c = lax.axis_index("c")
    s = lax.axis_index("s")
    w = s * 2 + c
    base = w * _RPW

    pltpu.sync_copy(ids_hbm.at[pl.ds(base, _RPW)], ids_v)
    pltpu.sync_copy(ts_hbm.at[pl.ds(base, _RPW)], ts_v)

    zeros16 = jnp.zeros((_L,), jnp.float32)

    def zbody(i, carry):
        acc_v[pl.ds(i * _L, _L)] = zeros16
        return carry

    lax.fori_loop(0, _SEG_PAD // _L, zbody, 0)

    lane = lax.broadcasted_iota(jnp.int32, (_L,), 0)

    def gbody(g, carry):
        ids16 = ids_v[pl.ds(g * _L, _L)]
        t16 = ts_v[pl.ds(g * _L, _L)]
        for d in (1, 2, 4, 8):
            fidx = jnp.minimum(lane + d, _L - 1)
            bidx = jnp.maximum(lane - d, 0)
            tf = t16.at[fidx].get(mode="promise_in_bounds")
            idf = ids16.at[fidx].get(mode="promise_in_bounds")
            tb = t16.at[bidx].get(mode="promise_in_bounds")
            idb = ids16.at[bidx].get(mode="promise_in_bounds")
            t16 = jnp.maximum(t16, jnp.where(idf == ids16, tf, 0.0))
            t16 = jnp.maximum(t16, jnp.where(idb == ids16, tb, 0.0))
        cur = plsc.load_gather(acc_v, [ids16])
        plsc.store_scatter(acc_v, [ids16], jnp.maximum(cur, t16))
        return carry

    lax.fori_loop(0, _RPW // _L, gbody, 0)

    pltpu.sync_copy(acc_v, out_hbm.at[pl.ds(w * _SEG_PAD, _SEG_PAD)])


_sc_partial = functools.partial(
    pl.kernel,
    out_type=jax.ShapeDtypeStruct((_NW * _SEG_PAD,), jnp.float32),
    mesh=plsc.VectorSubcoreMesh(core_axis_name="c", subcore_axis_name="s"),
    compiler_params=pltpu.CompilerParams(needs_layout_passes=False),
    scratch_types=[
        pltpu.VMEM((_RPW,), jnp.int32),
        pltpu.VMEM((_RPW,), jnp.float32),
        pltpu.VMEM((_SEG_PAD,), jnp.float32),
    ],
)(_sc_partial_body)


def _sc_fold_body(parts_hbm, out_hbm, buf_v, acc_v, sem):
    c = lax.axis_index("c")
    s = lax.axis_index("s")
    w = s * 2 + c
    base = w * _SPW

    descs = [
        pltpu.async_copy(
            parts_hbm.at[pl.ds(r * _SEG_PAD + base, _SPW)],
            buf_v.at[pl.ds(r * _SPW, _SPW)], sem)
        for r in range(_NW)
    ]
    for d in descs:
        d.wait()

    def fbody(i, carry):
        v = buf_v[pl.ds(i * _L, _L)]
        for r in range(1, _NW):
            v = jnp.maximum(v, buf_v[pl.ds(r * _SPW + i * _L, _L)])
        acc_v[pl.ds(i * _L, _L)] = v
        return carry

    lax.fori_loop(0, _SPW // _L, fbody, 0)

    pltpu.sync_copy(acc_v, out_hbm.at[pl.ds(base, _SPW)])


_sc_fold = functools.partial(
    pl.kernel,
    out_type=jax.ShapeDtypeStruct((_SEG_PAD,), jnp.float32),
    mesh=plsc.VectorSubcoreMesh(core_axis_name="c", subcore_axis_name="s"),
    compiler_params=pltpu.CompilerParams(needs_layout_passes=False),
    scratch_types=[
        pltpu.VMEM((_NW * _SPW,), jnp.float32),
        pltpu.VMEM((_SPW,), jnp.float32),
        pltpu.SemaphoreType.DMA,
    ],
)(_sc_fold_body)


def _tc_body(ids_ref, ts_ref, msg_ref, smax_ref, agg_ref, wsum_ref):
    j = pl.program_id(0)

    id_row = ids_ref[0]
    t_row = ts_ref[0]

    base0 = jnp.minimum((jnp.min(id_row) // 8) * 8, _SEG_PAD - _W)
    nw = (jnp.max(id_row) - base0) // _W + 1

    seg_iota = lax.broadcasted_iota(jnp.int32, (_W, 1), 0)

    @pl.when(j == 0)
    def _init():
        wsum_ref[...] = jnp.zeros_like(wsum_ref)
        agg_ref[...] = jnp.zeros_like(agg_ref)

    msg = msg_ref[...]
    ones_col = jnp.ones((_B, 1), jnp.float32)

    def body(w, carry):
        start = base0 + w * _W
        base = jnp.minimum(start, _SEG_PAD - _W)
        rel = jnp.where(id_row >= start, id_row - base, -1)
        eq = rel == seg_iota
        onehot = jnp.where(eq, 1.0, 0.0)
        smax_win = smax_ref[pl.ds(base, _W), :]
        last_t = jnp.sum(onehot * smax_win, axis=0, keepdims=True)
        arg = jnp.minimum(_BETA * (t_row - last_t), 0.0)
        w_row = jnp.exp(arg)
        wo = onehot * w_row
        wsum_ref[pl.ds(base, _W), :] += jnp.dot(
            wo, ones_col, preferred_element_type=jnp.float32)
        agg_ref[pl.ds(base, _W), :] += jnp.dot(
            wo, msg, preferred_element_type=jnp.float32)
        return carry

    lax.fori_loop(0, nw, body, 0)

    @pl.when(j == _NB - 1)
    def _finish():
        wsum = wsum_ref[...]
        inv = jnp.where(wsum > 0.0, 1.0 / wsum, 0.0)
        agg_ref[...] = agg_ref[...] * inv


@functools.partial(jax.jit, static_argnames=("interpret",))
def _run(node_ids, messages, timestamps, interpret=False):
    ids = node_ids.astype(jnp.int32)
    parts = _sc_partial(ids, timestamps)
    seg_max_col = _sc_fold(parts).reshape(_SEG_PAD, 1)

    ids3 = ids.reshape(_NB, 1, _B)
    ts3 = timestamps.reshape(_NB, 1, _B)

    out = pl.pallas_call(
        _tc_body,
        grid=(_NB,),
        in_specs=[
            pl.BlockSpec((1, 1, _B), lambda j: (j, 0, 0)),
            pl.BlockSpec((1, 1, _B), lambda j: (j, 0, 0)),
            pl.BlockSpec((_B, _D), lambda j: (j, 0)),
            pl.BlockSpec((_SEG_PAD, 1), lambda j: (0, 0)),
        ],
        out_specs=[
            pl.BlockSpec((_SEG_PAD, _D), lambda j: (0, 0)),
            pl.BlockSpec((_SEG_PAD, 1), lambda j: (0, 0)),
        ],
        out_shape=[
            jax.ShapeDtypeStruct((_SEG_PAD, _D), jnp.float32),
            jax.ShapeDtypeStruct((_SEG_PAD, 1), jnp.float32),
        ],
        interpret=interpret,
    )(ids3, ts3, messages, seg_max_col)

    agg, wsum = out
    return (agg[:_S], seg_max_col[:_S, 0], wsum[:_S, 0] > 0.0)


def kernel(node_ids, messages, timestamps):
    return _run(node_ids, messages, timestamps)
